# baseline (speedup 1.0000x reference)
import math

import jax
import jax.numpy as jnp
from jax import lax
from jax.experimental import pallas as pl
from jax.experimental.pallas import tpu as pltpu
from jax.experimental.pallas import tpu_sc as plsc

D_MODEL = 64
SCALE = math.sqrt(D_MODEL)

_NC = 2
_NS = 16
_NW = _NC * _NS
_LANES = 16

_A = 4096
_BD = 200
_AT = _A // 128
_BT = _BD // 8

_CB = 2
_N_CH = _BD // _CB


def _emb_body(xb_hbm, lut_hbm, o_hbm,
              idx_v, rows0, rows1, ot0, ot1, gsem0, gsem1, osem0, osem1):
    wid = lax.axis_index("s") * _NC + lax.axis_index("c")
    c = wid

    for bt in range(_BT):
        pltpu.sync_copy(xb_hbm.at[bt, c], idx_v.at[bt])

    rows = (rows0, rows1)
    otile = (ot0, ot1)
    gsem = (gsem0, gsem1)
    osem = (osem0, osem1)

    def g_fire(ci, b):
        for j in range(_CB):
            babs = ci * _CB + j
            pltpu.async_copy(
                lut_hbm.at[idx_v.at[babs // 8, babs % 8]],
                rows[b].at[pl.ds(j * 128, 128)],
                gsem[b])

    def g_drain(b):
        for j in range(_CB):
            pltpu.make_async_copy(
                lut_hbm.at[idx_v.at[0, 0]],
                rows[b].at[pl.ds(j * 128, 128)],
                gsem[b]).wait()

    def o_start(ci, b):
        pltpu.async_copy(otile[b], o_hbm.at[pl.ds(ci * _CB, _CB), :, c],
                         osem[b])

    def o_wait(b):
        pltpu.make_async_copy(otile[b], o_hbm.at[pl.ds(0, _CB), :, c],
                              osem[b]).wait()

    iotav = jax.lax.broadcasted_iota(jnp.int32, (16,), 0)

    def transpose(b):
        r_ref = rows[b]
        t_ref = otile[b]

        def lg_body(lg, carry):
            lgo = lg * 16
            base0 = iotav + lgo
            for bb in range(_CB):
                idx0 = base0 + bb * 128
                for d in range(D_MODEL):
                    idx1 = jnp.broadcast_to(jnp.int32(d), (16,))
                    v = plsc.load_gather(r_ref, [idx0, idx1])
                    t_ref[bb, d // 8, d % 8, pl.ds(lgo, 16)] = v * SCALE
            return carry

        lax.fori_loop(0, 8, lg_body, 0)

    g_fire(0, 0)
    g_fire(1, 1)

    def pair_body(p, carry):
        for b in range(2):
            ci = 2 * p + b
            g_drain(b)
            pl.when(p >= 1)(lambda: o_wait(b))
            transpose(b)
            pl.when(p <= (_N_CH // 2) - 2)(lambda: g_fire(ci + 2, b))
            o_start(ci, b)
        return carry

    lax.fori_loop(0, _N_CH // 2, pair_body, 0)

    o_wait(0)
    o_wait(1)


@jax.jit
def _emb(xb, lut):
    mesh = plsc.VectorSubcoreMesh(core_axis_name="c", subcore_axis_name="s")
    fn = pl.kernel(
        _emb_body,
        out_type=jax.ShapeDtypeStruct((_BD, 8, _AT, 8, 128), jnp.float32),
        mesh=mesh,
        scratch_types=[
            pltpu.VMEM((_BT, 8, 128), jnp.int32),
            pltpu.VMEM((_CB * 128, D_MODEL), jnp.float32),
            pltpu.VMEM((_CB * 128, D_MODEL), jnp.float32),
            pltpu.VMEM((_CB, 8, 8, 128), jnp.float32),
            pltpu.VMEM((_CB, 8, 8, 128), jnp.float32),
            pltpu.SemaphoreType.DMA,
            pltpu.SemaphoreType.DMA,
            pltpu.SemaphoreType.DMA,
            pltpu.SemaphoreType.DMA,
        ],
        compiler_params=pltpu.CompilerParams(
            use_tc_tiling_on_sc=False, needs_layout_passes=False),
    )
    return fn(xb, lut)


def kernel(x, lut):
    xb = x.reshape(32, 128, _BT, 8).transpose(2, 0, 3, 1)
    o = _emb(xb, lut)
    return o.transpose(2, 4, 0, 1, 3).reshape(_A, _BD, D_MODEL)


# speedup vs baseline: 1.1425x; 1.1425x over previous
import math

import jax
import jax.numpy as jnp
from jax import lax
from jax.experimental import pallas as pl
from jax.experimental.pallas import tpu as pltpu
from jax.experimental.pallas import tpu_sc as plsc

D_MODEL = 64
SCALE = math.sqrt(D_MODEL)

_NC = 2
_NS = 16
_NW = _NC * _NS

_A = 4096
_BD = 200
_AT = _A // 128
_BT = _BD // 8

_RPAD = 65
_CB = 2
_N_CH = _BD // _CB


def _emb_body(xb_hbm, lut_hbm, o_hbm,
              idx_v, rows0, rows1, rpad, ot0, ot1,
              gsem0, gsem1, osem0, osem1):
    wid = lax.axis_index("s") * _NC + lax.axis_index("c")
    c = wid

    for bt in range(_BT):
        pltpu.sync_copy(xb_hbm.at[bt, c], idx_v.at[bt])

    rows = (rows0, rows1)
    otile = (ot0, ot1)
    gsem = (gsem0, gsem1)
    osem = (osem0, osem1)

    def g_fire(ci, b):
        for j in range(_CB):
            babs = ci * _CB + j
            pltpu.async_copy(
                lut_hbm.at[idx_v.at[babs // 8, babs % 8]],
                rows[b].at[pl.ds(j * 128, 128)],
                gsem[b])

    def g_drain(b):
        for j in range(_CB):
            pltpu.make_async_copy(
                lut_hbm.at[idx_v.at[0, 0]],
                rows[b].at[pl.ds(j * 128, 128)],
                gsem[b]).wait()

    def o_start(ci, b):
        pltpu.async_copy(otile[b], o_hbm.at[pl.ds(ci * _CB, _CB), :, c],
                         osem[b])

    def o_wait(b):
        pltpu.make_async_copy(otile[b], o_hbm.at[pl.ds(0, _CB), :, c],
                              osem[b]).wait()

    iotav = jax.lax.broadcasted_iota(jnp.int32, (16,), 0)

    def transpose(b):
        r_ref = rows[b]

        def stage_body(rr, carry):
            r0 = rr * 8
            for k in range(8):
                for dg in range(D_MODEL // 16):
                    s = pl.ds(dg * 16, 16)
                    rpad[r0 + k, s] = r_ref[r0 + k, s]
            return carry

        lax.fori_loop(0, _CB * 16, stage_body, 0)
        t_ref = otile[b]

        def lg_body(lg, carry):
            lgo = lg * 16
            base0 = iotav + lgo
            for bb in range(_CB):
                idx0 = base0 + bb * 128
                for d in range(D_MODEL):
                    idx1 = jnp.broadcast_to(jnp.int32(d), (16,))
                    v = plsc.load_gather(rpad, [idx0, idx1])
                    t_ref[bb, d // 8, d % 8, pl.ds(lgo, 16)] = v * SCALE
            return carry

        lax.fori_loop(0, 8, lg_body, 0)

    g_fire(0, 0)
    g_fire(1, 1)

    def pair_body(p, carry):
        for b in range(2):
            ci = 2 * p + b
            g_drain(b)
            pl.when(p >= 1)(lambda: o_wait(b))
            transpose(b)
            pl.when(p <= (_N_CH // 2) - 2)(lambda: g_fire(ci + 2, b))
            o_start(ci, b)
        return carry

    lax.fori_loop(0, _N_CH // 2, pair_body, 0)

    o_wait(0)
    o_wait(1)


@jax.jit
def _emb(xb, lut):
    mesh = plsc.VectorSubcoreMesh(core_axis_name="c", subcore_axis_name="s")
    fn = pl.kernel(
        _emb_body,
        out_type=jax.ShapeDtypeStruct((_BD, 8, _AT, 8, 128), jnp.float32),
        mesh=mesh,
        scratch_types=[
            pltpu.VMEM((_BT, 8, 128), jnp.int32),
            pltpu.VMEM((_CB * 128, D_MODEL), jnp.float32),
            pltpu.VMEM((_CB * 128, D_MODEL), jnp.float32),
            pltpu.VMEM((_CB * 128, _RPAD), jnp.float32),
            pltpu.VMEM((_CB, 8, 8, 128), jnp.float32),
            pltpu.VMEM((_CB, 8, 8, 128), jnp.float32),
            pltpu.SemaphoreType.DMA,
            pltpu.SemaphoreType.DMA,
            pltpu.SemaphoreType.DMA,
            pltpu.SemaphoreType.DMA,
        ],
        compiler_params=pltpu.CompilerParams(
            use_tc_tiling_on_sc=False, needs_layout_passes=False),
    )
    return fn(xb, lut)


def kernel(x, lut):
    xb = x.reshape(32, 128, _BT, 8).transpose(2, 0, 3, 1)
    o = _emb(xb, lut)
    return o.transpose(2, 4, 0, 1, 3).reshape(_A, _BD, D_MODEL)


# speedup vs baseline: 1.7078x; 1.4948x over previous
import math

import jax
import jax.numpy as jnp
from jax import lax
from jax.experimental import pallas as pl
from jax.experimental.pallas import tpu as pltpu
from jax.experimental.pallas import tpu_sc as plsc

D_MODEL = 64
SCALE = math.sqrt(D_MODEL)

_NC = 2
_NS = 16
_NW = _NC * _NS

_A = 4096
_BD = 200
_AT = _A // 128
_BT = _BD // 8

_LP = 136
_CB = 2
_N_CH = _BD // _CB


def _emb_body(xb_hbm, lut_hbm, o_hbm,
              idx_v, rows0, rows1, ot0, ot1,
              gsem0, gsem1, osem0, osem1):
    wid = lax.axis_index("s") * _NC + lax.axis_index("c")
    c = wid

    for bt in range(_BT):
        pltpu.sync_copy(xb_hbm.at[bt, c], idx_v.at[bt])

    rows = (rows0, rows1)
    otile = (ot0, ot1)
    gsem = (gsem0, gsem1)
    osem = (osem0, osem1)

    def g_fire(ci, b):
        for j in range(_CB):
            babs = ci * _CB + j
            pltpu.async_copy(
                lut_hbm.at[idx_v.at[babs // 8, babs % 8]],
                rows[b].at[pl.ds(j * 128, 128)],
                gsem[b])

    def g_drain(b):
        for j in range(_CB):
            pltpu.make_async_copy(
                lut_hbm.at[idx_v.at[0, 0]],
                rows[b].at[pl.ds(j * 128, 128)],
                gsem[b]).wait()

    def o_start(ci, b):
        for bb in range(_CB):
            for t in range(8):
                pltpu.async_copy(
                    otile[b].at[pl.ds((bb * 64 + t * 8), 8), pl.ds(0, 128)],
                    o_hbm.at[ci * _CB + bb, t, c],
                    osem[b])

    def o_wait(b):
        for _ in range(_CB * 8):
            pltpu.make_async_copy(
                otile[b].at[pl.ds(0, 8), pl.ds(0, 128)],
                o_hbm.at[0, 0, c],
                osem[b]).wait()

    iotav = jax.lax.broadcasted_iota(jnp.int32, (16,), 0)
    base = [[(bb * 64 + dg * 16) + iotav
             for dg in range(4)] for bb in range(_CB)]

    def transpose(b):
        r_ref = rows[b]
        t_ref = otile[b]

        def l_body(l, carry):
            lv = jnp.broadcast_to(l, (16,))
            for bb in range(_CB):
                row = bb * 128 + l
                for dg in range(4):
                    v = r_ref[row, pl.ds(dg * 16, 16)]
                    plsc.store_scatter(t_ref, [base[bb][dg], lv], v * SCALE)
            return carry

        lax.fori_loop(0, 128, l_body, 0)

    g_fire(0, 0)
    g_fire(1, 1)

    def pair_body(p, carry):
        for b in range(2):
            ci = 2 * p + b
            g_drain(b)
            pl.when(p >= 1)(lambda: o_wait(b))
            transpose(b)
            pl.when(p <= (_N_CH // 2) - 2)(lambda: g_fire(ci + 2, b))
            o_start(ci, b)
        return carry

    lax.fori_loop(0, _N_CH // 2, pair_body, 0)

    o_wait(0)
    o_wait(1)


@jax.jit
def _emb(xb, lut):
    mesh = plsc.VectorSubcoreMesh(core_axis_name="c", subcore_axis_name="s")
    fn = pl.kernel(
        _emb_body,
        out_type=jax.ShapeDtypeStruct((_BD, 8, _AT, 8, 128), jnp.float32),
        mesh=mesh,
        scratch_types=[
            pltpu.VMEM((_BT, 8, 128), jnp.int32),
            pltpu.VMEM((_CB * 128, D_MODEL), jnp.float32),
            pltpu.VMEM((_CB * 128, D_MODEL), jnp.float32),
            pltpu.VMEM((_CB * 64, _LP), jnp.float32),
            pltpu.VMEM((_CB * 64, _LP), jnp.float32),
            pltpu.SemaphoreType.DMA,
            pltpu.SemaphoreType.DMA,
            pltpu.SemaphoreType.DMA,
            pltpu.SemaphoreType.DMA,
        ],
        compiler_params=pltpu.CompilerParams(
            use_tc_tiling_on_sc=False, needs_layout_passes=False),
    )
    return fn(xb, lut)


def kernel(x, lut):
    xb = x.reshape(32, 128, _BT, 8).transpose(2, 0, 3, 1)
    o = _emb(xb, lut)
    return o.transpose(2, 4, 0, 1, 3).reshape(_A, _BD, D_MODEL)


# speedup vs baseline: 1.7260x; 1.0106x over previous
import math

import jax
import jax.numpy as jnp
from jax import lax
from jax.experimental import pallas as pl
from jax.experimental.pallas import tpu as pltpu
from jax.experimental.pallas import tpu_sc as plsc

D_MODEL = 64
SCALE = math.sqrt(D_MODEL)

_NC = 2
_NS = 16
_NW = _NC * _NS

_A = 4096
_BD = 200
_AT = _A // 128
_BT = _BD // 8

_LP = 136
_CB = 2
_N_CH = _BD // _CB


def _emb_body(xb_hbm, lut_hbm, o_hbm,
              idx_v, rows0, rows1, ot0, ot1,
              gsem0, gsem1, osem0, osem1):
    wid = lax.axis_index("s") * _NC + lax.axis_index("c")
    c = wid

    for bt in range(_BT):
        pltpu.sync_copy(xb_hbm.at[bt, c], idx_v.at[bt])

    rows = (rows0, rows1)
    otile = (ot0, ot1)
    gsem = (gsem0, gsem1)
    osem = (osem0, osem1)

    def g_fire(ci, b):
        for j in range(_CB):
            babs = ci * _CB + j
            pltpu.async_copy(
                lut_hbm.at[idx_v.at[babs // 8, babs % 8]],
                rows[b].at[pl.ds(j * 128, 128)],
                gsem[b])

    def g_drain(b):
        for j in range(_CB):
            pltpu.make_async_copy(
                lut_hbm.at[idx_v.at[0, 0]],
                rows[b].at[pl.ds(j * 128, 128)],
                gsem[b]).wait()

    def o_start(ci, b):
        for bb in range(_CB):
            for t in range(8):
                pltpu.async_copy(
                    otile[b].at[pl.ds((bb * 64 + t * 8), 8), pl.ds(0, 128)],
                    o_hbm.at[ci * _CB + bb, t, c],
                    osem[b])

    def o_wait(b):
        for _ in range(_CB * 8):
            pltpu.make_async_copy(
                otile[b].at[pl.ds(0, 8), pl.ds(0, 128)],
                o_hbm.at[0, 0, c],
                osem[b]).wait()

    iotav = jax.lax.broadcasted_iota(jnp.int32, (16,), 0)
    base = [[(bb * 64 + dg * 16) + iotav
             for dg in range(4)] for bb in range(_CB)]

    def transpose(b):
        r_ref = rows[b]
        t_ref = otile[b]

        def l_body(l, carry):
            lv = jnp.broadcast_to(l, (16,))
            for bb in range(_CB):
                row = bb * 128 + l
                for dg in range(4):
                    v = r_ref[row, pl.ds(dg * 16, 16)]
                    plsc.store_scatter(t_ref, [base[bb][dg], lv], v * SCALE)
            return carry

        lax.fori_loop(0, 128, l_body, 0, unroll=4)

    g_fire(0, 0)
    g_fire(1, 1)

    def pair_body(p, carry):
        for b in range(2):
            ci = 2 * p + b
            g_drain(b)
            pl.when(p >= 1)(lambda: o_wait(b))
            transpose(b)
            pl.when(p <= (_N_CH // 2) - 2)(lambda: g_fire(ci + 2, b))
            o_start(ci, b)
        return carry

    lax.fori_loop(0, _N_CH // 2, pair_body, 0)

    o_wait(0)
    o_wait(1)


@jax.jit
def _emb(xb, lut):
    mesh = plsc.VectorSubcoreMesh(core_axis_name="c", subcore_axis_name="s")
    fn = pl.kernel(
        _emb_body,
        out_type=jax.ShapeDtypeStruct((_BD, 8, _AT, 8, 128), jnp.float32),
        mesh=mesh,
        scratch_types=[
            pltpu.VMEM((_BT, 8, 128), jnp.int32),
            pltpu.VMEM((_CB * 128, D_MODEL), jnp.float32),
            pltpu.VMEM((_CB * 128, D_MODEL), jnp.float32),
            pltpu.VMEM((_CB * 64, _LP), jnp.float32),
            pltpu.VMEM((_CB * 64, _LP), jnp.float32),
            pltpu.SemaphoreType.DMA,
            pltpu.SemaphoreType.DMA,
            pltpu.SemaphoreType.DMA,
            pltpu.SemaphoreType.DMA,
        ],
        compiler_params=pltpu.CompilerParams(
            use_tc_tiling_on_sc=False, needs_layout_passes=False),
    )
    return fn(xb, lut)


def kernel(x, lut):
    xb = x.reshape(32, 128, _BT, 8).transpose(2, 0, 3, 1)
    o = _emb(xb, lut)
    return o.transpose(2, 4, 0, 1, 3).reshape(_A, _BD, D_MODEL)
